# baseline (speedup 1.0000x reference)
import jax
import jax.numpy as jnp
from jax.experimental import pallas as pl
from jax.experimental.pallas import tpu as pltpu

NBINS = 100
ROWS_PER_BLOCK = 8


def _norm_kernel(t_ref, mean_ref, std_ref, x_ref, o_ref):
    i = pl.program_id(0)
    for r in range(ROWS_PER_BLOCK):
        row = i * ROWS_PER_BLOCK + r
        tb = (t_ref[row] * NBINS).astype(jnp.int32)
        tb = jnp.where(tb == NBINS, NBINS - 1, tb)
        m = mean_ref[tb]
        s = std_ref[tb]
        o_ref[r] = (x_ref[r] - m) * (1.0 / s)


def kernel(x_t, t, data_mean, data_std):
    B = x_t.shape[0]
    nb = B // ROWS_PER_BLOCK
    grid_spec = pltpu.PrefetchScalarGridSpec(
        num_scalar_prefetch=3,
        grid=(nb,),
        in_specs=[pl.BlockSpec((ROWS_PER_BLOCK, 4, 256, 256),
                               lambda i, *_: (i, 0, 0, 0))],
        out_specs=pl.BlockSpec((ROWS_PER_BLOCK, 4, 256, 256),
                               lambda i, *_: (i, 0, 0, 0)),
    )
    return pl.pallas_call(
        _norm_kernel,
        grid_spec=grid_spec,
        out_shape=jax.ShapeDtypeStruct(x_t.shape, x_t.dtype),
        compiler_params=pltpu.CompilerParams(
            dimension_semantics=("arbitrary",),
        ),
    )(t, data_mean, data_std, x_t)
